# Initial kernel scaffold; baseline (speedup 1.0000x reference)
import jax
import jax.numpy as jnp
from jax.experimental import pallas as pl


def kernel(img, transfos):
    raise NotImplementedError("write your pallas kernel here")


# speedup vs baseline: 1.0081x; 1.0081x over previous
import functools

import jax
import jax.numpy as jnp
from jax import lax
from jax.experimental import pallas as pl
from jax.experimental.pallas import tpu as pltpu
from jax.experimental.pallas import tpu_sc as plsc

B, D, H, W = 4, 96, 96, 96
N = D * H * W
TOT = B * N
NTILES = 32
SPT = TOT // NTILES
CHUNK = 1536
NC = SPT // CHUNK
FMAX = float(W - 1)


def _coords(transfos):
    zs = jnp.linspace(-1.0, 1.0, D)
    ys = jnp.linspace(-1.0, 1.0, H)
    xs = jnp.linspace(-1.0, 1.0, W)
    mz, my, mx = jnp.meshgrid(zs, ys, xs, indexing="ij")
    grid = jnp.stack([mx.reshape(-1), my.reshape(-1), mz.reshape(-1),
                      jnp.ones(N, dtype=jnp.float32)], axis=0)
    points = jnp.einsum("bij,jn->bin", transfos, grid)
    coeff = jnp.float32((W - 1) / 2.0)
    ix = coeff * points[:, 0, :] + coeff
    iy = coeff * points[:, 1, :] + coeff
    iz = coeff * points[:, 2, :] + coeff
    return ix.reshape(-1), iy.reshape(-1), iz.reshape(-1)


@functools.partial(
    pl.kernel,
    out_type=jax.ShapeDtypeStruct((TOT,), jnp.float32),
    mesh=plsc.VectorSubcoreMesh(core_axis_name="c", subcore_axis_name="s"),
    compiler_params=pltpu.CompilerParams(needs_layout_passes=False),
    scratch_types=[
        pltpu.VMEM((3 * CHUNK,), jnp.float32),
        pltpu.VMEM((8 * CHUNK,), jnp.int32),
        pltpu.VMEM((8 * CHUNK,), jnp.float32),
        pltpu.VMEM((8 * CHUNK,), jnp.float32),
        pltpu.VMEM((CHUNK,), jnp.float32),
        pltpu.SemaphoreType.DMA,
    ],
)
def _interp(img_hbm, ix_hbm, iy_hbm, iz_hbm, out_hbm,
            crd_v, idx_v, w_v, vals_v, out_v, sem):
    wid = lax.axis_index("s") * 2 + lax.axis_index("c")
    bbase = (wid // (NTILES // B)) * N

    def chunk_body(g, carry):
        start = wid * SPT + g * CHUNK
        pltpu.sync_copy(ix_hbm.at[pl.ds(start, CHUNK)], crd_v.at[pl.ds(0, CHUNK)])
        pltpu.sync_copy(iy_hbm.at[pl.ds(start, CHUNK)], crd_v.at[pl.ds(CHUNK, CHUNK)])
        pltpu.sync_copy(iz_hbm.at[pl.ds(start, CHUNK)], crd_v.at[pl.ds(2 * CHUNK, CHUNK)])
        bbv = jnp.full((16,), bbase, jnp.int32)

        def pass1(v, c2):
            base = v * 16
            ix = crd_v[pl.ds(base, 16)]
            iy = crd_v[pl.ds(CHUNK + base, 16)]
            iz = crd_v[pl.ds(2 * CHUNK + base, 16)]
            valid = ((ix >= 0.0) & (ix <= FMAX) & (iy >= 0.0) & (iy <= FMAX)
                     & (iz >= 0.0) & (iz <= FMAX))
            vf = jnp.where(valid, jnp.float32(1.0), jnp.float32(0.0))
            ix0 = jnp.clip(ix, 0.0, FMAX).astype(jnp.int32)
            iy0 = jnp.clip(iy, 0.0, FMAX).astype(jnp.int32)
            iz0 = jnp.clip(iz, 0.0, FMAX).astype(jnp.int32)
            fx = ix - ix0.astype(jnp.float32)
            fy = iy - iy0.astype(jnp.float32)
            fz = iz - iz0.astype(jnp.float32)
            ux = (jnp.abs(fx - 1.0) * vf, jnp.abs(fx) * vf)
            uy = (jnp.abs(fy - 1.0), jnp.abs(fy))
            uz = (jnp.abs(fz - 1.0), jnp.abs(fz))
            ax = (ix0, jnp.minimum(ix0 + 1, W - 1))
            ay = (iy0 * W, jnp.minimum(iy0 + 1, H - 1) * W)
            az = (iz0 * (H * W) + bbv,
                  jnp.minimum(iz0 + 1, D - 1) * (H * W) + bbv)
            for c in range(8):
                dx, dy, dz = c >> 2, (c >> 1) & 1, c & 1
                idx_v[pl.ds(c * CHUNK + base, 16)] = ax[dx] + ay[dy] + az[dz]
                w_v[pl.ds(c * CHUNK + base, 16)] = ux[dx] * (uy[dy] * uz[dz])
            return c2

        lax.fori_loop(0, CHUNK // 16, pass1, 0)
        pltpu.async_copy(img_hbm.at[idx_v], vals_v, sem).wait()

        def pass2(v, c2):
            base = v * 16
            acc = w_v[pl.ds(base, 16)] * vals_v[pl.ds(base, 16)]
            for c in range(1, 8):
                acc = acc + (w_v[pl.ds(c * CHUNK + base, 16)]
                             * vals_v[pl.ds(c * CHUNK + base, 16)])
            out_v[pl.ds(base, 16)] = acc
            return c2

        lax.fori_loop(0, CHUNK // 16, pass2, 0)
        pltpu.sync_copy(out_v, out_hbm.at[pl.ds(start, CHUNK)])
        return carry

    lax.fori_loop(0, NC, chunk_body, 0)


def kernel(img, transfos):
    img_flat = img.reshape(-1)
    ix, iy, iz = _coords(transfos)
    out = _interp(img_flat, ix, iy, iz)
    return out.reshape(img.shape)
